# Initial kernel scaffold; baseline (speedup 1.0000x reference)
import functools
import jax
import jax.numpy as jnp
from jax import lax
from jax.experimental import pallas as pl
from jax.experimental.pallas import tpu as pltpu
from jax.experimental.pallas import tpu_sc as plsc

N = 10000
E = 320000
C = 128


def _probe_sc(table, idx):
    mesh = plsc.VectorSubcoreMesh(core_axis_name="c", subcore_axis_name="s")
    NP = 10240
    RPW = 320
    CS = 1600
    CAP = 160

    @functools.partial(
        pl.kernel,
        out_type=jax.ShapeDtypeStruct((NP * C,), jnp.float32),
        mesh=mesh,
        scratch_types=[
            pltpu.VMEM((RPW * C,), jnp.float32),
            pltpu.VMEM((CS,), jnp.int32),
            pltpu.VMEM((CAP,), jnp.int32),
            pltpu.VMEM((CAP,), jnp.int32),
            pltpu.VMEM((CAP, C), jnp.float32),
            pltpu.SemaphoreType.DMA,
        ],
    )
    def k(tab_hbm, idx_hbm, out_hbm, acc, dbuf, ebuf, mbuf, rows, sem):
        wid = lax.axis_index("s") * 2 + lax.axis_index("c")
        lo = wid * RPW

        @pl.loop(0, RPW * C, step=16)
        def _(i):
            acc[pl.ds(i, 16)] = jnp.full((16,), -jnp.inf, jnp.float32)

        lane = lax.iota(jnp.int32, 16)

        def chunk_body(ci, pos0):
            base = ci * CS
            pltpu.sync_copy(idx_hbm.at[pl.ds(base, CS)], dbuf)

            def vstep(vi, pos):
                dv = dbuf[pl.ds(vi * 16, 16)]
                msk = (dv >= lo) & (dv < lo + RPW)
                eidv = base + vi * 16 + lane
                plsc.store_compressed(ebuf.at[pl.ds(pos, 16)], eidv, msk)
                plsc.store_compressed(mbuf.at[pl.ds(pos, 16)], dv, msk)
                cnt = jnp.sum(msk.astype(jnp.int32))
                pos = pos + cnt

                def flush():
                    pltpu.async_copy(tab_hbm.at[ebuf], rows, sem).wait()

                    def rloop(j, carry):
                        d = mbuf[j]
                        off = (d - lo) * C
                        for kk in range(C // 16):
                            a = acc[pl.ds(off + kk * 16, 16)]
                            r = rows[j, pl.ds(kk * 16, 16)]
                            acc[pl.ds(off + kk * 16, 16)] = jnp.maximum(a, r)
                        return carry

                    lax.fori_loop(0, pos, rloop, 0)

                pl.when(pos >= CAP - 16)(flush)
                return jnp.where(pos >= CAP - 16, 0, pos)

            return lax.fori_loop(0, CS // 16, vstep, pos0)

        lax.fori_loop(0, E // CS, chunk_body, jnp.int32(0))
        pltpu.sync_copy(acc, out_hbm.at[pl.ds(lo * C, RPW * C)])

    return k(table, idx)


def kernel(x, edge_index, edge_attr, params):
    dst = edge_index[1]
    out = _probe_sc(x, dst)
    return out.reshape(NP0, C)[:N] * 0.0 + x


NP0 = 10240


# speedup vs baseline: 1.2567x; 1.2567x over previous
import dataclasses
import functools

import jax
import jax.numpy as jnp
from jax import lax
from jax.experimental import pallas as pl
from jax.experimental.pallas import tpu as pltpu
from jax.experimental.pallas import tpu_sc as plsc

N = 10000
E = 320000
C = 128
NTR = 10496
RPT = 656
TRM = 255
NP = 10240
RPW = 320
CS = 1600
CAP = 160
GC = 400
EW = E // 32
SCH = 200
EPT = E // 32
ZB = 16

_HIGHEST = jax.lax.Precision.HIGHEST


def _sc_params():
    cp = pltpu.CompilerParams()
    if "needs_layout_passes" in pltpu.CompilerParams.__dataclass_fields__:
        cp = dataclasses.replace(cp, needs_layout_passes=False)
    return cp


def _mesh():
    return plsc.VectorSubcoreMesh(core_axis_name="c", subcore_axis_name="s")


def _sc_gather2(P, Q, dst, src):
    @functools.partial(
        pl.kernel,
        out_type=(jax.ShapeDtypeStruct((E, C), jnp.float32),
                  jax.ShapeDtypeStruct((E, C), jnp.float32)),
        mesh=_mesh(),
        compiler_params=_sc_params(),
        scratch_types=[
            pltpu.VMEM((GC,), jnp.int32),
            pltpu.VMEM((GC, C), jnp.float32),
            pltpu.SemaphoreType.DMA,
        ],
    )
    def k(p_hbm, q_hbm, dst_hbm, src_hbm, zp_hbm, zq_hbm, idxb, rowb, sem):
        wid = lax.axis_index("s") * 2 + lax.axis_index("c")
        base = wid * EW
        for tab, idx, out in ((p_hbm, dst_hbm, zp_hbm),
                              (q_hbm, src_hbm, zq_hbm)):
            @pl.loop(0, EW, step=GC)
            def _(kk, tab=tab, idx=idx, out=out):
                pltpu.sync_copy(idx.at[pl.ds(base + kk, GC)], idxb)
                pltpu.async_copy(tab.at[idxb], rowb, sem).wait()
                pltpu.sync_copy(rowb, out.at[pl.ds(base + kk, GC)])

    return k(P, Q, dst, src)


def _sc_sage(y, src, dstp):
    @functools.partial(
        pl.kernel,
        out_type=jax.ShapeDtypeStruct((2, NTR, C), jnp.float32),
        mesh=_mesh(),
        compiler_params=_sc_params(),
        scratch_types=[
            pltpu.VMEM_SHARED((NTR, C), jnp.float32),
            pltpu.VMEM((ZB, C), jnp.float32),
            pltpu.VMEM((SCH,), jnp.int32),
            pltpu.VMEM((SCH,), jnp.int32),
            pltpu.VMEM((SCH, C), jnp.float32),
            pltpu.SemaphoreType.DMA,
        ],
    )
    def k(y_hbm, src_hbm, dst_hbm, out_hbm, accS, zb, sidx, didx, rowb, sem):
        c = lax.axis_index("c")
        s = lax.axis_index("s")
        zrow = s * RPT

        @pl.loop(0, ZB)
        def _(i):
            @pl.loop(0, C, step=16)
            def _(j):
                zb[i, pl.ds(j, 16)] = jnp.zeros((16,), jnp.float32)

        @pl.loop(0, RPT, step=ZB)
        def _(q):
            pltpu.sync_copy(zb, accS.at[pl.ds(zrow + q, ZB)])
        plsc.subcore_barrier()

        base = c * (E // 2) + s * EPT

        @pl.loop(0, EPT, step=SCH)
        def _(kk):
            pltpu.sync_copy(src_hbm.at[pl.ds(base + kk, SCH)], sidx)
            pltpu.sync_copy(dst_hbm.at[pl.ds(base + kk, SCH)], didx)
            pltpu.async_copy(y_hbm.at[sidx], rowb, sem).wait()
            pltpu.sync_copy(rowb, accS.at[didx], add=True)

        plsc.subcore_barrier()
        pltpu.sync_copy(accS.at[pl.ds(zrow, RPT)],
                        out_hbm.at[c, pl.ds(zrow, RPT)])

    return k(y, src, dstp)


def _sc_count(cnt_rows, dst):
    @functools.partial(
        pl.kernel,
        out_type=jax.ShapeDtypeStruct((2, NTR, C), jnp.float32),
        mesh=_mesh(),
        compiler_params=_sc_params(),
        scratch_types=[
            pltpu.VMEM_SHARED((NTR, C), jnp.float32),
            pltpu.VMEM((ZB, C), jnp.float32),
            pltpu.VMEM((SCH,), jnp.int32),
            pltpu.VMEM((SCH, C), jnp.float32),
            pltpu.SemaphoreType.DMA,
        ],
    )
    def k(r_hbm, dst_hbm, out_hbm, accS, zb, didx, rowb, sem):
        c = lax.axis_index("c")
        s = lax.axis_index("s")
        zrow = s * RPT

        @pl.loop(0, ZB)
        def _(i):
            @pl.loop(0, C, step=16)
            def _(j):
                zb[i, pl.ds(j, 16)] = jnp.zeros((16,), jnp.float32)

        @pl.loop(0, RPT, step=ZB)
        def _(q):
            pltpu.sync_copy(zb, accS.at[pl.ds(zrow + q, ZB)])
        plsc.subcore_barrier()

        base = c * (E // 2) + s * EPT

        @pl.loop(0, EPT, step=SCH)
        def _(kk):
            pltpu.sync_copy(dst_hbm.at[pl.ds(base + kk, SCH)], didx)
            pltpu.sync_copy(r_hbm.at[pl.ds(base + kk, SCH)], rowb)
            pltpu.sync_copy(rowb, accS.at[didx], add=True)

        plsc.subcore_barrier()
        pltpu.sync_copy(accS.at[pl.ds(zrow, RPT)],
                        out_hbm.at[c, pl.ds(zrow, RPT)])

    return k(cnt_rows, dst)


def _sc_segmax(m, dst):
    @functools.partial(
        pl.kernel,
        out_type=jax.ShapeDtypeStruct((NP * C,), jnp.float32),
        mesh=_mesh(),
        compiler_params=_sc_params(),
        scratch_types=[
            pltpu.VMEM((RPW * C,), jnp.float32),
            pltpu.VMEM((CS,), jnp.int32),
            pltpu.VMEM((CAP + 16,), jnp.int32),
            pltpu.VMEM((CAP + 16,), jnp.int32),
            pltpu.VMEM((CAP + 16, C), jnp.float32),
            pltpu.SemaphoreType.DMA,
        ],
    )
    def k(m_hbm, dst_hbm, out_hbm, acc, dbuf, ebuf, mbuf, rows, sem):
        wid = lax.axis_index("s") * 2 + lax.axis_index("c")
        lo = wid * RPW

        @pl.loop(0, RPW * C, step=16)
        def _(i):
            acc[pl.ds(i, 16)] = jnp.full((16,), -jnp.inf, jnp.float32)

        @pl.loop(0, CAP + 16, step=16)
        def _(i):
            ebuf[pl.ds(i, 16)] = jnp.zeros((16,), jnp.int32)

        lane = lax.iota(jnp.int32, 16)

        def flush(pos):
            pltpu.async_copy(m_hbm.at[ebuf], rows, sem).wait()

            def rloop(j, carry):
                d = mbuf[pl.ds(j, 16)][0]
                off = (d - lo) * C
                for kk in range(C // 16):
                    a = acc[pl.ds(off + kk * 16, 16)]
                    r = rows[j, pl.ds(kk * 16, 16)]
                    acc[pl.ds(off + kk * 16, 16)] = jnp.maximum(a, r)
                return carry

            lax.fori_loop(0, pos, rloop, 0)

        def chunk_body(ci, pos0):
            base = ci * CS
            pltpu.sync_copy(dst_hbm.at[pl.ds(base, CS)], dbuf)

            def vstep(vi, pos):
                dv = dbuf[pl.ds(vi * 16, 16)]
                msk = (dv >= lo) & (dv < lo + RPW)
                eidv = base + vi * 16 + lane
                plsc.store_compressed(ebuf.at[pl.ds(pos, 16)], eidv, mask=msk)
                plsc.store_compressed(mbuf.at[pl.ds(pos, 16)], dv, mask=msk)
                pos = pos + jnp.sum(msk.astype(jnp.int32))
                pl.when(pos >= CAP)(lambda: flush(pos))
                return jnp.where(pos >= CAP, 0, pos)

            return lax.fori_loop(0, CS // 16, vstep, pos0)

        posf = lax.fori_loop(0, E // CS, chunk_body, jnp.int32(0))
        pl.when(posf > 0)(lambda: flush(posf))
        pltpu.sync_copy(acc, out_hbm.at[pl.ds(lo * C, RPW * C)])

    return k(m, dst)


def _bn_tc(t, g, b):
    mu = jnp.mean(t, axis=0, keepdims=True)
    va = jnp.mean((t - mu) * (t - mu), axis=0, keepdims=True)
    return (t - mu) * jax.lax.rsqrt(va + 1e-5) * g + b


def _dot(a, b):
    return jnp.dot(a, b, preferred_element_type=jnp.float32,
                   precision=_HIGHEST)


def _tc_node_pre(x, g0, b0, wds, b1s, wbs):
    def bn_body(x_r, g_r, b_r, h_r):
        h_r[...] = jax.nn.relu(_bn_tc(x_r[...], g_r[...], b_r[...]))

    h = pl.pallas_call(
        bn_body,
        out_shape=jax.ShapeDtypeStruct((N, C), jnp.float32),
    )(x, g0, b0)

    def mm_body(h_r, wd1, bb1, wb1, wd2, bb2, wb2, wd3, bb3, wb3,
                p1, q1, p2, q2, p3, q3):
        hh = h_r[...]
        for wd, bb, wb, p, q in ((wd1, bb1, wb1, p1, q1),
                                 (wd2, bb2, wb2, p2, q2),
                                 (wd3, bb3, wb3, p3, q3)):
            p[...] = _dot(hh, wd[...]) + bb[...]
            q[...] = _dot(hh, wb[...])

    BLKN = 2000
    wspec = pl.BlockSpec((C, C), lambda i: (0, 0))
    bspec = pl.BlockSpec((1, C), lambda i: (0, 0))
    nspec = pl.BlockSpec((BLKN, C), lambda i: (i, 0))
    return pl.pallas_call(
        mm_body,
        grid=(N // BLKN,),
        in_specs=[nspec, wspec, bspec, wspec, wspec, bspec, wspec,
                  wspec, bspec, wspec],
        out_specs=tuple(nspec for _ in range(6)),
        out_shape=tuple(jax.ShapeDtypeStruct((N, C), jnp.float32)
                        for _ in range(6)),
    )(h, wds[0], b1s[0], wbs[0], wds[1], b1s[1], wbs[1],
      wds[2], b1s[2], wbs[2])


def _tc_edge_prep(dst2, attr2):
    def body(d_r, a_r, of_r, ob_r):
        d = d_r[...]
        a = a_r[...]
        r = jax.lax.broadcasted_iota(jnp.int32, d.shape, 0)
        cc = jax.lax.broadcasted_iota(jnp.int32, d.shape, 1)
        trash = N + ((r * 128 + cc) & TRM)
        of_r[...] = jnp.where(a <= 0, d, trash)
        ob_r[...] = jnp.where(a >= 0, d, trash)

    return pl.pallas_call(
        body,
        out_shape=(jax.ShapeDtypeStruct(dst2.shape, jnp.int32),
                   jax.ShapeDtypeStruct(dst2.shape, jnp.int32)),
    )(dst2, attr2)


def _tc_cnt_rows(attr1):
    BLK = 8000

    def body(a_r, o_r):
        a = a_r[...]
        cc = jax.lax.broadcasted_iota(jnp.int32, (BLK, C), 1)
        f = (a <= 0).astype(jnp.float32)
        b = (a >= 0).astype(jnp.float32)
        o_r[...] = jnp.where(cc == 0, f, 0.0) + jnp.where(cc == 1, b, 0.0) \
            + jnp.where(cc == 2, 1.0, 0.0)

    return pl.pallas_call(
        body,
        grid=(E // BLK,),
        in_specs=[pl.BlockSpec((BLK, 1), lambda i: (i, 0))],
        out_specs=pl.BlockSpec((BLK, C), lambda i: (i, 0)),
        out_shape=jax.ShapeDtypeStruct((E, C), jnp.float32),
    )(attr1)


def _tc_edge_mm(zp, zq, w2, attr1, mode):
    BLK = 4000

    def body(zp_r, zq_r, w_r, a_r, o_r):
        z = jax.nn.relu(zp_r[...] + zq_r[...])
        mm = _dot(z, w_r[...])
        if mode == "f":
            ok = a_r[...] <= 0
            mm = jnp.where(ok, mm, -jnp.inf)
        elif mode == "b":
            ok = a_r[...] >= 0
            mm = jnp.where(ok, mm, -jnp.inf)
        o_r[...] = mm

    return pl.pallas_call(
        body,
        grid=(E // BLK,),
        in_specs=[
            pl.BlockSpec((BLK, C), lambda i: (i, 0)),
            pl.BlockSpec((BLK, C), lambda i: (i, 0)),
            pl.BlockSpec((C, C), lambda i: (0, 0)),
            pl.BlockSpec((BLK, 1), lambda i: (i, 0)),
        ],
        out_specs=pl.BlockSpec((BLK, C), lambda i: (i, 0)),
        out_shape=jax.ShapeDtypeStruct((E, C), jnp.float32),
    )(zp, zq, w2, attr1)


def _tc_postmax(mx, b2, g1, b1g, wl, wr, bl):
    def body(mx_r, b2_r, g_r, bg_r, wl_r, wr_r, bl_r, y_r, hr_r):
        mxv = mx_r[...]
        gmax = jnp.where(jnp.isfinite(mxv), mxv + b2_r[...], 0.0)
        h1 = jax.nn.relu(_bn_tc(gmax, g_r[...], bg_r[...]))
        y_r[...] = _dot(h1, wl_r[...])
        hr_r[...] = _dot(h1, wr_r[...]) + bl_r[...]

    return pl.pallas_call(
        body,
        out_shape=(jax.ShapeDtypeStruct((N, C), jnp.float32),
                   jax.ShapeDtypeStruct((N, C), jnp.float32)),
    )(mx, b2, g1, b1g, wl, wr, bl)


def _tc_combine_mid(a0, a1, c0, c1, hr, g2, b2g, wl, wr, bl):
    def bn_body(a0_r, a1_r, c0_r, c1_r, hr_r, g_r, b_r, h_r):
        s = a0_r[...] + a1_r[...]
        cnt = jnp.maximum(c0_r[...] + c1_r[...], 1.0)
        v = s / cnt + hr_r[...]
        h_r[...] = jax.nn.relu(_bn_tc(v, g_r[...], b_r[...]))

    h2 = pl.pallas_call(
        bn_body,
        out_shape=jax.ShapeDtypeStruct((N, C), jnp.float32),
    )(a0, a1, c0, c1, hr, g2, b2g)

    def mm_body(h_r, wl_r, wr_r, bl_r, y_r, o_r):
        hh = h_r[...]
        y_r[...] = _dot(hh, wl_r[...])
        o_r[...] = _dot(hh, wr_r[...]) + bl_r[...]

    return pl.pallas_call(
        mm_body,
        out_shape=(jax.ShapeDtypeStruct((N, C), jnp.float32),
                   jax.ShapeDtypeStruct((N, C), jnp.float32)),
    )(h2, wl, wr, bl)


def _tc_combine_final(a0, a1, c0, c1, hr, g3, b3g, acc_prev):
    def body(a0_r, a1_r, c0_r, c1_r, hr_r, g_r, b_r, p_r, o_r):
        s = a0_r[...] + a1_r[...]
        cnt = jnp.maximum(c0_r[...] + c1_r[...], 1.0)
        v = s / cnt + hr_r[...]
        o_r[...] = p_r[...] + _bn_tc(v, g_r[...], b_r[...])

    return pl.pallas_call(
        body,
        out_shape=jax.ShapeDtypeStruct((N, C), jnp.float32),
    )(a0, a1, c0, c1, hr, g3, b3g, acc_prev)


def kernel(x, edge_index, edge_attr, params):
    src = edge_index[0]
    dst = edge_index[1]
    attr1 = edge_attr.reshape(E, 1)

    wds, b1s, wbs = [], [], []
    for i in (1, 2, 3):
        w1 = params["ec%d" % i]["W1"]
        wds.append(w1[:C] - w1[C:])
        wbs.append(w1[C:])
        b1s.append(params["ec%d" % i]["b1"].reshape(1, C))

    p1, q1, p2, q2, p3, q3 = _tc_node_pre(
        x, params["bn0"]["g"].reshape(1, C), params["bn0"]["b"].reshape(1, C),
        wds, b1s, wbs)
    pq = {1: (p1, q1), 2: (p2, q2), 3: (p3, q3)}

    dstp_f, dstp_b = _tc_edge_prep(dst.reshape(E // 128, 128),
                                   edge_attr.reshape(E // 128, 128))
    dstp = {1: dstp_f.reshape(E), 2: dstp_b.reshape(E), 3: dst}
    modes = {1: "f", 2: "b", 3: "all"}

    cacc = _sc_count(_tc_cnt_rows(attr1), dst)
    ccols = {i: (cacc[0, :N, i - 1:i], cacc[1, :N, i - 1:i]) for i in (1, 2, 3)}

    out = None
    for i in (1, 2, 3):
        P, Q = pq[i]
        zp, zq = _sc_gather2(P, Q, dst, src)
        m = _tc_edge_mm(zp, zq, params["ec%d" % i]["W2"], attr1, modes[i])
        mx = _sc_segmax(m, dst).reshape(NP, C)[:N]
        y2, hr2 = _tc_postmax(
            mx, params["ec%d" % i]["b2"].reshape(1, C),
            params["bn1%d" % i]["g"].reshape(1, C),
            params["bn1%d" % i]["b"].reshape(1, C),
            params["s2%d" % i]["Wl"], params["s2%d" % i]["Wr"],
            params["s2%d" % i]["bl"].reshape(1, C))
        c0, c1 = ccols[i]
        acc2 = _sc_sage(y2, src, dstp[i])
        y3, hr3 = _tc_combine_mid(
            acc2[0, :N], acc2[1, :N], c0, c1, hr2,
            params["bn2%d" % i]["g"].reshape(1, C),
            params["bn2%d" % i]["b"].reshape(1, C),
            params["s3%d" % i]["Wl"], params["s3%d" % i]["Wr"],
            params["s3%d" % i]["bl"].reshape(1, C))
        acc3 = _sc_sage(y3, src, dstp[i])
        if out is None:
            prev = jnp.zeros((N, C), jnp.float32)
        else:
            prev = out
        out = _tc_combine_final(
            acc3[0, :N], acc3[1, :N], c0, c1, hr3,
            params["bn3%d" % i]["g"].reshape(1, C),
            params["bn3%d" % i]["b"].reshape(1, C), prev)

    return out
